# baseline (speedup 1.0000x reference)
import functools

import jax
import jax.numpy as jnp
from jax import lax
from jax.experimental import pallas as pl
from jax.experimental.pallas import tpu as pltpu
from jax.experimental.pallas import tpu_sc as plsc

_K = 128


@functools.lru_cache(maxsize=None)
def _sc_kernels(N, E, H):
    info = plsc.get_sparse_core_info()
    NC, NS = info.num_cores, info.num_subcores
    NW = NC * NS

    assert E % _K == 0, "edge count must be padded to a multiple of 128"
    CH = E // _K
    q, r = divmod(CH, NW)
    NPAIR = q // 2

    NA = -(-(N + 1) // (NS * 8)) * NS * 8
    RPT = NA // NS

    mesh = plsc.VectorSubcoreMesh(core_axis_name="c", subcore_axis_name="s")

    @functools.partial(
        pl.kernel,
        mesh=mesh,
        out_type=jax.ShapeDtypeStruct((NC * NA, H), jnp.float32),
        scratch_types=[
            pltpu.VMEM((2, _K), jnp.int32),
            pltpu.VMEM((2, _K), jnp.int32),
            pltpu.VMEM((2, _K, H), jnp.float32),
            pltpu.VMEM_SHARED((NA, H), jnp.float32),
            pltpu.SemaphoreType.DMA,
            pltpu.SemaphoreType.DMA,
            pltpu.SemaphoreType.DMA,
            pltpu.SemaphoreType.DMA,
            pltpu.SemaphoreType.DMA,
            pltpu.SemaphoreType.DMA,
        ],
    )
    def agg_kernel(xs, srce, dste, zeros, out, srcb, dstb, rowsb, acc,
                   g0, g1, ss0, ss1, sd0, sd1):
        cid = lax.axis_index("c")
        sid = lax.axis_index("s")
        wid = cid * NS + sid
        nchw = q + jnp.where(wid < r, 1, 0)
        c0 = wid * q + jnp.minimum(wid, r)

        pltpu.sync_copy(zeros.at[pl.ds(sid * RPT, RPT)],
                        acc.at[pl.ds(sid * RPT, RPT)])
        plsc.subcore_barrier()

        gsem = (g0, g1)
        ssem = (ss0, ss1)
        dsem = (sd0, sd1)

        def ebase(j):
            return (c0 + j) * _K

        def load_src(j, s):
            pltpu.async_copy(srce.at[pl.ds(ebase(j), _K)], srcb.at[s],
                             ssem[s])

        def load_dst(j, s):
            pltpu.async_copy(dste.at[pl.ds(ebase(j), _K)], dstb.at[s],
                             dsem[s])

        def wait_src(s):
            pltpu.make_async_copy(srce.at[pl.ds(0, _K)], srcb.at[s],
                                  ssem[s]).wait()

        def wait_dst(s):
            pltpu.make_async_copy(dste.at[pl.ds(0, _K)], dstb.at[s],
                                  dsem[s]).wait()

        def gather(s):
            pltpu.async_copy(xs.at[srcb.at[s]], rowsb.at[s], gsem[s])

        def wait_gather(s):
            pltpu.make_async_copy(xs.at[srcb.at[s]], rowsb.at[s],
                                  gsem[s]).wait()

        load_src(0, 0)
        load_src(1, 1)
        load_dst(0, 0)
        load_dst(1, 1)
        wait_src(0)
        gather(0)

        def chunk_body(j, s):
            o = 1 - s

            wait_gather(s)

            @pl.when(j + 1 < nchw)
            def _():
                wait_src(o)
                gather(o)

            @pl.when(j + 2 < nchw)
            def _():
                load_src(j + 2, s)

            wait_dst(s)
            pltpu.sync_copy(rowsb.at[s], acc.at[dstb.at[s]], add=True)

            @pl.when(j + 2 < nchw)
            def _():
                load_dst(j + 2, s)

        def pair(jp, carry):
            chunk_body(2 * jp, 0)
            chunk_body(2 * jp + 1, 1)
            return carry

        lax.fori_loop(0, NPAIR, pair, 0)
        if q % 2:
            chunk_body(q - 1, (q - 1) % 2)
        if r:
            @pl.when(wid < r)
            def _():
                chunk_body(q, q % 2)

        plsc.subcore_barrier()
        pltpu.sync_copy(acc.at[pl.ds(sid * RPT, RPT)],
                        out.at[pl.ds(cid * NA + sid * RPT, RPT)])

    @functools.partial(
        pl.kernel,
        mesh=mesh,
        out_type=jax.ShapeDtypeStruct((NW * NA,), jnp.float32),
        compiler_params=pltpu.CompilerParams(needs_layout_passes=False),
        scratch_types=[
            pltpu.VMEM((2, _K), jnp.int32),
            pltpu.VMEM((NA,), jnp.float32),
            pltpu.SemaphoreType.DMA,
            pltpu.SemaphoreType.DMA,
        ],
    )
    def hist_kernel(dste, zeros1, out, dstb, hist, sd0, sd1):
        cid = lax.axis_index("c")
        sid = lax.axis_index("s")
        wid = cid * NS + sid
        nchw = q + jnp.where(wid < r, 1, 0)
        c0 = wid * q + jnp.minimum(wid, r)

        pltpu.sync_copy(zeros1, hist)

        dsem = (sd0, sd1)

        def load_dst(j, s):
            pltpu.async_copy(dste.at[pl.ds((c0 + j) * _K, _K)], dstb.at[s],
                             dsem[s])

        def wait_dst(s):
            pltpu.make_async_copy(dste.at[pl.ds(0, _K)], dstb.at[s],
                                  dsem[s]).wait()

        lanes = lax.iota(jnp.int32, 16)
        ones16 = jnp.ones((16,), jnp.float32)

        def count16(d):
            for k in range(16):
                plsc.addupdate_scatter(hist, [d], ones16, mask=lanes == k)

        def chunk_body(j, s):
            wait_dst(s)
            for v in range(_K // 16):
                count16(dstb[s, pl.ds(v * 16, 16)])

            @pl.when(j + 2 < nchw)
            def _():
                load_dst(j + 2, s)

        load_dst(0, 0)
        load_dst(1, 1)

        def pair(jp, carry):
            chunk_body(2 * jp, 0)
            chunk_body(2 * jp + 1, 1)
            return carry

        lax.fori_loop(0, NPAIR, pair, 0)
        if q % 2:
            chunk_body(q - 1, (q - 1) % 2)
        if r:
            @pl.when(wid < r)
            def _():
                chunk_body(q, q % 2)

        pltpu.sync_copy(hist, out.at[pl.ds(wid * NA, NA)])

    return agg_kernel, hist_kernel, NA, NC, NW


def _pre_body(x_ref, w_ref, degp_ref, xs_ref, dinv_ref):
    deg = 1.0 + jnp.sum(degp_ref[...], axis=0)
    dinv = lax.rsqrt(deg)
    dinv_ref[...] = dinv
    xs_ref[...] = dinv * jnp.dot(x_ref[...], w_ref[...],
                                 preferred_element_type=jnp.float32)


def _mid_body(p_ref, xs_ref, dinv_ref, b_ref, w_ref, o_ref):
    dinv = dinv_ref[...]
    h = jnp.maximum(dinv * (p_ref[0] + p_ref[1] + xs_ref[...]) + b_ref[...],
                    0.0)
    o_ref[...] = dinv * jnp.dot(h, w_ref[...],
                                preferred_element_type=jnp.float32)


def _final_body(n_nodes, ncols, p_ref, xs_ref, dinv_ref, b_ref, wl_ref,
                bl_ref, o_ref, acc_ref):
    i = pl.program_id(0)

    @pl.when(i == 0)
    def _():
        acc_ref[...] = jnp.zeros_like(acc_ref)

    h = dinv_ref[...] * (p_ref[0] + p_ref[1] + xs_ref[...]) + b_ref[...]
    acc_ref[...] += jnp.sum(h, axis=0, keepdims=True)

    @pl.when(i == pl.num_programs(0) - 1)
    def _():
        pooled = acc_ref[...] * (1.0 / n_nodes)
        logits = jnp.dot(pooled, wl_ref[...],
                         preferred_element_type=jnp.float32) + bl_ref[...]
        col = lax.broadcasted_iota(jnp.int32, logits.shape, 1)
        valid = col < ncols
        mx = jnp.max(jnp.where(valid, logits, -jnp.inf), axis=1,
                     keepdims=True)
        ez = jnp.where(valid, jnp.exp(logits - mx), 0.0)
        o_ref[...] = ez / jnp.sum(ez, axis=1, keepdims=True)


def kernel(x, edge_index, W0, b0, W1, b1, W2, b2, Wlin, blin):
    N, D = x.shape
    H = W0.shape[1]
    C = Wlin.shape[1]
    E = edge_index.shape[1]
    src = edge_index[0]
    dst = edge_index[1]
    if E % _K:
        pad = _K - E % _K
        src = jnp.concatenate([src, jnp.zeros((pad,), src.dtype)])
        dst = jnp.concatenate([dst, jnp.full((pad,), N, dst.dtype)])
        E += pad

    agg_kernel, hist_kernel, NA, NC, NW = _sc_kernels(N, E, H)

    zerosH = jnp.zeros((NA, H), jnp.float32)
    zeros1 = jnp.zeros((NA,), jnp.float32)

    degp = hist_kernel(dst, zeros1).reshape(NW, NA, 1)

    BR = 1000 if N % 1000 == 0 else 8
    grid = (N // BR,)
    b0r, b1r, b2r = (b.reshape(1, H) for b in (b0, b1, b2))
    wl_pad = jnp.zeros((H, 128), jnp.float32).at[:, :C].set(Wlin)
    bl_pad = jnp.zeros((1, 128), jnp.float32).at[:, :C].set(blin)

    xs0, dinv = pl.pallas_call(
        _pre_body,
        grid=grid,
        in_specs=[
            pl.BlockSpec((BR, D), lambda i: (i, 0)),
            pl.BlockSpec((D, H), lambda i: (0, 0)),
            pl.BlockSpec((NW, BR, 1), lambda i: (0, i, 0)),
        ],
        out_specs=[
            pl.BlockSpec((BR, H), lambda i: (i, 0)),
            pl.BlockSpec((BR, 1), lambda i: (i, 0)),
        ],
        out_shape=[
            jax.ShapeDtypeStruct((N, H), jnp.float32),
            jax.ShapeDtypeStruct((N, 1), jnp.float32),
        ],
    )(x, W0, degp)

    def mid(parts, xs, b, w):
        return pl.pallas_call(
            _mid_body,
            grid=grid,
            in_specs=[
                pl.BlockSpec((NC, BR, H), lambda i: (0, i, 0)),
                pl.BlockSpec((BR, H), lambda i: (i, 0)),
                pl.BlockSpec((BR, 1), lambda i: (i, 0)),
                pl.BlockSpec((1, H), lambda i: (0, 0)),
                pl.BlockSpec((H, H), lambda i: (0, 0)),
            ],
            out_specs=pl.BlockSpec((BR, H), lambda i: (i, 0)),
            out_shape=jax.ShapeDtypeStruct((N, H), jnp.float32),
        )(parts, xs, dinv, b, w)

    p1 = agg_kernel(xs0, src, dst, zerosH).reshape(NC, NA, H)
    xs1 = mid(p1, xs0, b0r, W1)
    p2 = agg_kernel(xs1, src, dst, zerosH).reshape(NC, NA, H)
    xs2 = mid(p2, xs1, b1r, W2)
    p3 = agg_kernel(xs2, src, dst, zerosH).reshape(NC, NA, H)

    out = pl.pallas_call(
        functools.partial(_final_body, N, C),
        grid=grid,
        in_specs=[
            pl.BlockSpec((NC, BR, H), lambda i: (0, i, 0)),
            pl.BlockSpec((BR, H), lambda i: (i, 0)),
            pl.BlockSpec((BR, 1), lambda i: (i, 0)),
            pl.BlockSpec((1, H), lambda i: (0, 0)),
            pl.BlockSpec((H, 128), lambda i: (0, 0)),
            pl.BlockSpec((1, 128), lambda i: (0, 0)),
        ],
        out_specs=pl.BlockSpec((1, 128), lambda i: (0, 0)),
        out_shape=jax.ShapeDtypeStruct((1, 128), jnp.float32),
        scratch_shapes=[pltpu.VMEM((1, 128), jnp.float32)],
    )(p3, xs2, dinv, b2r, wl_pad, bl_pad)

    return out[:, :C]


# speedup vs baseline: 1.3945x; 1.3945x over previous
import functools

import jax
import jax.numpy as jnp
from jax import lax
from jax.experimental import pallas as pl
from jax.experimental.pallas import tpu as pltpu
from jax.experimental.pallas import tpu_sc as plsc

_K = 128


@functools.lru_cache(maxsize=None)
def _sc_kernels(N, E, H):
    info = plsc.get_sparse_core_info()
    NC, NS = info.num_cores, info.num_subcores
    NW = NC * NS

    assert E % _K == 0, "edge count must be padded to a multiple of 128"
    CH = E // _K
    q, r = divmod(CH, NW)
    NPAIR = q // 2

    NA = -(-(N + 1) // (NS * 8)) * NS * 8
    RPT = NA // NS
    NH = -(-(N + 1) // 1024) * 1024

    mesh = plsc.VectorSubcoreMesh(core_axis_name="c", subcore_axis_name="s")

    @functools.partial(
        pl.kernel,
        mesh=mesh,
        out_type=jax.ShapeDtypeStruct((NC * NA, H), jnp.float32),
        scratch_types=[
            pltpu.VMEM((2, _K), jnp.int32),
            pltpu.VMEM((2, _K), jnp.int32),
            pltpu.VMEM((2, _K, H), jnp.float32),
            pltpu.VMEM_SHARED((NA, H), jnp.float32),
            pltpu.SemaphoreType.DMA,
            pltpu.SemaphoreType.DMA,
            pltpu.SemaphoreType.DMA,
            pltpu.SemaphoreType.DMA,
            pltpu.SemaphoreType.DMA,
            pltpu.SemaphoreType.DMA,
        ],
    )
    def agg_kernel(xs, srce, dste, zeros, out, srcb, dstb, rowsb, acc,
                   g0, g1, ss0, ss1, sd0, sd1):
        cid = lax.axis_index("c")
        sid = lax.axis_index("s")
        wid = cid * NS + sid
        nchw = q + jnp.where(wid < r, 1, 0)
        c0 = wid * q + jnp.minimum(wid, r)

        pltpu.sync_copy(zeros.at[pl.ds(sid * RPT, RPT)],
                        acc.at[pl.ds(sid * RPT, RPT)])
        plsc.subcore_barrier()

        gsem = (g0, g1)
        ssem = (ss0, ss1)
        dsem = (sd0, sd1)

        def ebase(j):
            return (c0 + j) * _K

        def load_src(j, s):
            pltpu.async_copy(srce.at[pl.ds(ebase(j), _K)], srcb.at[s],
                             ssem[s])

        def load_dst(j, s):
            pltpu.async_copy(dste.at[pl.ds(ebase(j), _K)], dstb.at[s],
                             dsem[s])

        def wait_src(s):
            pltpu.make_async_copy(srce.at[pl.ds(0, _K)], srcb.at[s],
                                  ssem[s]).wait()

        def wait_dst(s):
            pltpu.make_async_copy(dste.at[pl.ds(0, _K)], dstb.at[s],
                                  dsem[s]).wait()

        def gather(s):
            pltpu.async_copy(xs.at[srcb.at[s]], rowsb.at[s], gsem[s])

        def wait_gather(s):
            pltpu.make_async_copy(xs.at[srcb.at[s]], rowsb.at[s],
                                  gsem[s]).wait()

        load_src(0, 0)
        load_src(1, 1)
        load_dst(0, 0)
        load_dst(1, 1)
        wait_src(0)
        gather(0)

        def chunk_body(j, s):
            o = 1 - s

            wait_gather(s)

            @pl.when(j + 1 < nchw)
            def _():
                wait_src(o)
                gather(o)

            @pl.when(j + 2 < nchw)
            def _():
                load_src(j + 2, s)

            wait_dst(s)
            pltpu.sync_copy(rowsb.at[s], acc.at[dstb.at[s]], add=True)

            @pl.when(j + 2 < nchw)
            def _():
                load_dst(j + 2, s)

        def pair(jp, carry):
            chunk_body(2 * jp, 0)
            chunk_body(2 * jp + 1, 1)
            return carry

        lax.fori_loop(0, NPAIR, pair, 0)
        if q % 2:
            chunk_body(q - 1, (q - 1) % 2)
        if r:
            @pl.when(wid < r)
            def _():
                chunk_body(q, q % 2)

        plsc.subcore_barrier()
        pltpu.sync_copy(acc.at[pl.ds(sid * RPT, RPT)],
                        out.at[pl.ds(cid * NA + sid * RPT, RPT)])

    @functools.partial(
        pl.kernel,
        mesh=mesh,
        out_type=jax.ShapeDtypeStruct((NW * NH,), jnp.float32),
        compiler_params=pltpu.CompilerParams(needs_layout_passes=False),
        scratch_types=[
            pltpu.VMEM((2, _K), jnp.int32),
            pltpu.VMEM((NH,), jnp.float32),
            pltpu.SemaphoreType.DMA,
            pltpu.SemaphoreType.DMA,
        ],
    )
    def hist_kernel(dste, zeros1, out, dstb, hist, sd0, sd1):
        cid = lax.axis_index("c")
        sid = lax.axis_index("s")
        wid = cid * NS + sid
        nchw = q + jnp.where(wid < r, 1, 0)
        c0 = wid * q + jnp.minimum(wid, r)

        pltpu.sync_copy(zeros1, hist)

        dsem = (sd0, sd1)

        def load_dst(j, s):
            pltpu.async_copy(dste.at[pl.ds((c0 + j) * _K, _K)], dstb.at[s],
                             dsem[s])

        def wait_dst(s):
            pltpu.make_async_copy(dste.at[pl.ds(0, _K)], dstb.at[s],
                                  dsem[s]).wait()

        lanes = lax.iota(jnp.int32, 16)
        ones16 = jnp.ones((16,), jnp.float32)

        def count16(d):
            for k in range(16):
                plsc.addupdate_scatter(hist, [d], ones16, mask=lanes == k)

        def chunk_body(j, s):
            wait_dst(s)
            for v in range(_K // 16):
                count16(dstb[s, pl.ds(v * 16, 16)])

            @pl.when(j + 2 < nchw)
            def _():
                load_dst(j + 2, s)

        load_dst(0, 0)
        load_dst(1, 1)

        def pair(jp, carry):
            chunk_body(2 * jp, 0)
            chunk_body(2 * jp + 1, 1)
            return carry

        lax.fori_loop(0, NPAIR, pair, 0)
        if q % 2:
            chunk_body(q - 1, (q - 1) % 2)
        if r:
            @pl.when(wid < r)
            def _():
                chunk_body(q, q % 2)

        pltpu.sync_copy(hist, out.at[pl.ds(wid * NH, NH)])

    return agg_kernel, hist_kernel, NA, NH, NC, NW


def _pre_body(x_ref, w_ref, degp_ref, xs_ref, dinv_ref):
    deg = 1.0 + jnp.sum(degp_ref[...], axis=0)
    dinv = lax.rsqrt(deg).reshape(-1, 1)
    dinv_ref[...] = dinv
    xs_ref[...] = dinv * jnp.dot(x_ref[...], w_ref[...],
                                 preferred_element_type=jnp.float32)


def _mid_body(p_ref, xs_ref, dinv_ref, b_ref, w_ref, o_ref):
    dinv = dinv_ref[...]
    h = jnp.maximum(dinv * (p_ref[0] + p_ref[1] + xs_ref[...]) + b_ref[...],
                    0.0)
    o_ref[...] = dinv * jnp.dot(h, w_ref[...],
                                preferred_element_type=jnp.float32)


def _final_body(n_nodes, ncols, p_ref, xs_ref, dinv_ref, b_ref, wl_ref,
                bl_ref, o_ref, acc_ref):
    i = pl.program_id(0)

    @pl.when(i == 0)
    def _():
        acc_ref[...] = jnp.zeros_like(acc_ref)

    h = dinv_ref[...] * (p_ref[0] + p_ref[1] + xs_ref[...]) + b_ref[...]
    acc_ref[...] += jnp.sum(h, axis=0, keepdims=True)

    @pl.when(i == pl.num_programs(0) - 1)
    def _():
        pooled = acc_ref[...] * (1.0 / n_nodes)
        logits = jnp.dot(pooled, wl_ref[...],
                         preferred_element_type=jnp.float32) + bl_ref[...]
        col = lax.broadcasted_iota(jnp.int32, logits.shape, 1)
        valid = col < ncols
        mx = jnp.max(jnp.where(valid, logits, -jnp.inf), axis=1,
                     keepdims=True)
        ez = jnp.where(valid, jnp.exp(logits - mx), 0.0)
        o_ref[...] = ez / jnp.sum(ez, axis=1, keepdims=True)


def kernel(x, edge_index, W0, b0, W1, b1, W2, b2, Wlin, blin):
    N, D = x.shape
    H = W0.shape[1]
    C = Wlin.shape[1]
    E = edge_index.shape[1]
    src = edge_index[0]
    dst = edge_index[1]
    if E % _K:
        pad = _K - E % _K
        src = jnp.concatenate([src, jnp.zeros((pad,), src.dtype)])
        dst = jnp.concatenate([dst, jnp.full((pad,), N, dst.dtype)])
        E += pad

    agg_kernel, hist_kernel, NA, NH, NC, NW = _sc_kernels(N, E, H)

    zerosH = jnp.zeros((NA, H), jnp.float32)
    zeros1 = jnp.zeros((NH,), jnp.float32)

    degp = hist_kernel(dst, zeros1).reshape(NW, NH)

    BR = 1000 if N % 1000 == 0 else 8
    grid = (N // BR,)
    b0r, b1r, b2r = (b.reshape(1, H) for b in (b0, b1, b2))
    wl_pad = jnp.zeros((H, 128), jnp.float32).at[:, :C].set(Wlin)
    bl_pad = jnp.zeros((1, 128), jnp.float32).at[:, :C].set(blin)

    BRP = 1024
    xs0, dinv = pl.pallas_call(
        _pre_body,
        grid=(NH // BRP,),
        in_specs=[
            pl.BlockSpec((BRP, D), lambda i: (i, 0)),
            pl.BlockSpec((D, H), lambda i: (0, 0)),
            pl.BlockSpec((NW, BRP), lambda i: (0, i)),
        ],
        out_specs=[
            pl.BlockSpec((BRP, H), lambda i: (i, 0)),
            pl.BlockSpec((BRP, 1), lambda i: (i, 0)),
        ],
        out_shape=[
            jax.ShapeDtypeStruct((N, H), jnp.float32),
            jax.ShapeDtypeStruct((N, 1), jnp.float32),
        ],
    )(x, W0, degp)

    def mid(parts, xs, b, w):
        return pl.pallas_call(
            _mid_body,
            grid=grid,
            in_specs=[
                pl.BlockSpec((NC, BR, H), lambda i: (0, i, 0)),
                pl.BlockSpec((BR, H), lambda i: (i, 0)),
                pl.BlockSpec((BR, 1), lambda i: (i, 0)),
                pl.BlockSpec((1, H), lambda i: (0, 0)),
                pl.BlockSpec((H, H), lambda i: (0, 0)),
            ],
            out_specs=pl.BlockSpec((BR, H), lambda i: (i, 0)),
            out_shape=jax.ShapeDtypeStruct((N, H), jnp.float32),
        )(parts, xs, dinv, b, w)

    p1 = agg_kernel(xs0, src, dst, zerosH).reshape(NC, NA, H)
    xs1 = mid(p1, xs0, b0r, W1)
    p2 = agg_kernel(xs1, src, dst, zerosH).reshape(NC, NA, H)
    xs2 = mid(p2, xs1, b1r, W2)
    p3 = agg_kernel(xs2, src, dst, zerosH).reshape(NC, NA, H)

    out = pl.pallas_call(
        functools.partial(_final_body, N, C),
        grid=grid,
        in_specs=[
            pl.BlockSpec((NC, BR, H), lambda i: (0, i, 0)),
            pl.BlockSpec((BR, H), lambda i: (i, 0)),
            pl.BlockSpec((BR, 1), lambda i: (i, 0)),
            pl.BlockSpec((1, H), lambda i: (0, 0)),
            pl.BlockSpec((H, 128), lambda i: (0, 0)),
            pl.BlockSpec((1, 128), lambda i: (0, 0)),
        ],
        out_specs=pl.BlockSpec((1, 128), lambda i: (0, 0)),
        out_shape=jax.ShapeDtypeStruct((1, 128), jnp.float32),
        scratch_shapes=[pltpu.VMEM((1, 128), jnp.float32)],
    )(p3, xs2, dinv, b2r, wl_pad, bl_pad)

    return out[:, :C]


# speedup vs baseline: 1.4070x; 1.0090x over previous
import functools

import jax
import jax.numpy as jnp
from jax import lax
from jax.experimental import pallas as pl
from jax.experimental.pallas import tpu as pltpu
from jax.experimental.pallas import tpu_sc as plsc

_K = 128


@functools.lru_cache(maxsize=None)
def _sc_kernels(N, E, H):
    info = plsc.get_sparse_core_info()
    NC, NS = info.num_cores, info.num_subcores
    NW = NC * NS

    assert E % _K == 0, "edge count must be padded to a multiple of 128"
    CH = E // _K
    q, r = divmod(CH, NW)
    NPAIR = q // 2

    NA = -(-(N + 1) // (NS * 8)) * NS * 8
    RPT = NA // NS
    NH = -(-(N + 1) // 1024) * 1024

    mesh = plsc.VectorSubcoreMesh(core_axis_name="c", subcore_axis_name="s")

    @functools.partial(
        pl.kernel,
        mesh=mesh,
        out_type=jax.ShapeDtypeStruct((NC * NA, H), jnp.float32),
        scratch_types=[
            pltpu.VMEM((2, _K), jnp.int32),
            pltpu.VMEM((4, _K), jnp.int32),
            pltpu.VMEM((2, _K, H), jnp.float32),
            pltpu.VMEM_SHARED((NA, H), jnp.float32),
            pltpu.SemaphoreType.DMA,
            pltpu.SemaphoreType.DMA,
            pltpu.SemaphoreType.DMA,
            pltpu.SemaphoreType.DMA,
            pltpu.SemaphoreType.DMA,
            pltpu.SemaphoreType.DMA,
            pltpu.SemaphoreType.DMA,
            pltpu.SemaphoreType.DMA,
            pltpu.SemaphoreType.DMA,
            pltpu.SemaphoreType.DMA,
        ],
    )
    def agg_kernel(xs, srce, dste, zeros, out, srcb, dstb, rowsb, acc,
                   g0, g1, ss0, ss1, sd0, sd1, sd2, sd3, sc0, sc1):
        cid = lax.axis_index("c")
        sid = lax.axis_index("s")
        wid = cid * NS + sid
        nchw = q + jnp.where(wid < r, 1, 0)
        c0 = wid * q + jnp.minimum(wid, r)

        gsem = (g0, g1)
        ssem = (ss0, ss1)
        dsem = (sd0, sd1, sd2, sd3)
        csem = (sc0, sc1)

        def ebase(j):
            return (c0 + j) * _K

        def load_src(j, s):
            pltpu.async_copy(srce.at[pl.ds(ebase(j), _K)], srcb.at[s],
                             ssem[s])

        def load_dst(j, s4):
            pltpu.async_copy(dste.at[pl.ds(ebase(j), _K)], dstb.at[s4],
                             dsem[s4])

        def wait_src(s):
            pltpu.make_async_copy(srce.at[pl.ds(0, _K)], srcb.at[s],
                                  ssem[s]).wait()

        def wait_dst(s4):
            pltpu.make_async_copy(dste.at[pl.ds(0, _K)], dstb.at[s4],
                                  dsem[s4]).wait()

        def gather(s):
            pltpu.async_copy(xs.at[srcb.at[s]], rowsb.at[s], gsem[s])

        def wait_gather(s):
            pltpu.make_async_copy(xs.at[srcb.at[s]], rowsb.at[s],
                                  gsem[s]).wait()

        def scatter(s, s4):
            pltpu.async_copy(rowsb.at[s], acc.at[dstb.at[s4]], csem[s],
                             add=True)

        def wait_scatter(s):
            pltpu.make_async_copy(rowsb.at[s], acc.at[dstb.at[s]],
                                  csem[s]).wait()

        load_src(0, 0)
        load_src(1, 1)
        for jj in range(3):
            load_dst(jj, jj)
        wait_src(0)
        gather(0)
        pltpu.sync_copy(zeros.at[pl.ds(sid * RPT, RPT)],
                        acc.at[pl.ds(sid * RPT, RPT)])
        plsc.subcore_barrier()

        def chunk_body(j, s, s4):
            o = 1 - s

            wait_gather(s)
            wait_dst(s4)
            scatter(s, s4)

            @pl.when(j + 1 < nchw)
            def _():
                @pl.when(j >= 1)
                def _():
                    wait_scatter(o)
                wait_src(o)
                gather(o)

                @pl.when(j + 3 < nchw)
                def _():
                    load_dst(j + 3, (s4 + 3) % 4)

            @pl.when(j + 2 < nchw)
            def _():
                load_src(j + 2, s)

        def chunk4(jq, carry):
            j = 4 * jq
            chunk_body(j, 0, 0)
            chunk_body(j + 1, 1, 1)
            chunk_body(j + 2, 0, 2)
            chunk_body(j + 3, 1, 3)
            return carry

        lax.fori_loop(0, q // 4, chunk4, 0)
        for j in range(q - q % 4, q):
            chunk_body(j, j % 2, j % 4)
        if r:
            @pl.when(wid < r)
            def _():
                chunk_body(q, q % 2, q % 4)

        wait_scatter(0)
        wait_scatter(1)
        plsc.subcore_barrier()
        pltpu.sync_copy(acc.at[pl.ds(sid * RPT, RPT)],
                        out.at[pl.ds(cid * NA + sid * RPT, RPT)])

    @functools.partial(
        pl.kernel,
        mesh=mesh,
        out_type=jax.ShapeDtypeStruct((NW * NH,), jnp.float32),
        compiler_params=pltpu.CompilerParams(needs_layout_passes=False),
        scratch_types=[
            pltpu.VMEM((2, _K), jnp.int32),
            pltpu.VMEM((NH,), jnp.float32),
            pltpu.SemaphoreType.DMA,
            pltpu.SemaphoreType.DMA,
        ],
    )
    def hist_kernel(dste, zeros1, out, dstb, hist, sd0, sd1):
        cid = lax.axis_index("c")
        sid = lax.axis_index("s")
        wid = cid * NS + sid
        nchw = q + jnp.where(wid < r, 1, 0)
        c0 = wid * q + jnp.minimum(wid, r)

        pltpu.sync_copy(zeros1, hist)

        dsem = (sd0, sd1)

        def load_dst(j, s):
            pltpu.async_copy(dste.at[pl.ds((c0 + j) * _K, _K)], dstb.at[s],
                             dsem[s])

        def wait_dst(s):
            pltpu.make_async_copy(dste.at[pl.ds(0, _K)], dstb.at[s],
                                  dsem[s]).wait()

        lanes = lax.iota(jnp.int32, 16)
        ones16 = jnp.ones((16,), jnp.float32)

        def count16(d):
            for k in range(16):
                plsc.addupdate_scatter(hist, [d], ones16, mask=lanes == k)

        def chunk_body(j, s):
            wait_dst(s)
            for v in range(_K // 16):
                count16(dstb[s, pl.ds(v * 16, 16)])

            @pl.when(j + 2 < nchw)
            def _():
                load_dst(j + 2, s)

        load_dst(0, 0)
        load_dst(1, 1)

        def pair(jp, carry):
            chunk_body(2 * jp, 0)
            chunk_body(2 * jp + 1, 1)
            return carry

        lax.fori_loop(0, NPAIR, pair, 0)
        if q % 2:
            chunk_body(q - 1, (q - 1) % 2)
        if r:
            @pl.when(wid < r)
            def _():
                chunk_body(q, q % 2)

        pltpu.sync_copy(hist, out.at[pl.ds(wid * NH, NH)])

    return agg_kernel, hist_kernel, NA, NH, NC, NW


def _pre_body(x_ref, w_ref, degp_ref, xs_ref, dinv_ref):
    deg = 1.0 + jnp.sum(degp_ref[...], axis=0)
    dinv = lax.rsqrt(deg).reshape(-1, 1)
    dinv_ref[...] = dinv
    xs_ref[...] = dinv * jnp.dot(x_ref[...], w_ref[...],
                                 preferred_element_type=jnp.float32)


def _mid_body(p_ref, xs_ref, dinv_ref, b_ref, w_ref, o_ref):
    dinv = dinv_ref[...]
    h = jnp.maximum(dinv * (p_ref[0] + p_ref[1] + xs_ref[...]) + b_ref[...],
                    0.0)
    o_ref[...] = dinv * jnp.dot(h, w_ref[...],
                                preferred_element_type=jnp.float32)


def _final_body(n_nodes, ncols, p_ref, xs_ref, dinv_ref, b_ref, wl_ref,
                bl_ref, o_ref, acc_ref):
    i = pl.program_id(0)

    @pl.when(i == 0)
    def _():
        acc_ref[...] = jnp.zeros_like(acc_ref)

    h = dinv_ref[...] * (p_ref[0] + p_ref[1] + xs_ref[...]) + b_ref[...]
    acc_ref[...] += jnp.sum(h, axis=0, keepdims=True)

    @pl.when(i == pl.num_programs(0) - 1)
    def _():
        pooled = acc_ref[...] * (1.0 / n_nodes)
        logits = jnp.dot(pooled, wl_ref[...],
                         preferred_element_type=jnp.float32) + bl_ref[...]
        col = lax.broadcasted_iota(jnp.int32, logits.shape, 1)
        valid = col < ncols
        mx = jnp.max(jnp.where(valid, logits, -jnp.inf), axis=1,
                     keepdims=True)
        ez = jnp.where(valid, jnp.exp(logits - mx), 0.0)
        o_ref[...] = ez / jnp.sum(ez, axis=1, keepdims=True)


def kernel(x, edge_index, W0, b0, W1, b1, W2, b2, Wlin, blin):
    N, D = x.shape
    H = W0.shape[1]
    C = Wlin.shape[1]
    E = edge_index.shape[1]
    src = edge_index[0]
    dst = edge_index[1]
    if E % _K:
        pad = _K - E % _K
        src = jnp.concatenate([src, jnp.zeros((pad,), src.dtype)])
        dst = jnp.concatenate([dst, jnp.full((pad,), N, dst.dtype)])
        E += pad

    agg_kernel, hist_kernel, NA, NH, NC, NW = _sc_kernels(N, E, H)

    zerosH = jnp.zeros((NA, H), jnp.float32)
    zeros1 = jnp.zeros((NH,), jnp.float32)

    degp = hist_kernel(dst, zeros1).reshape(NW, NH)

    BR = 1000 if N % 1000 == 0 else 8
    grid = (N // BR,)
    b0r, b1r, b2r = (b.reshape(1, H) for b in (b0, b1, b2))
    wl_pad = jnp.zeros((H, 128), jnp.float32).at[:, :C].set(Wlin)
    bl_pad = jnp.zeros((1, 128), jnp.float32).at[:, :C].set(blin)

    BRP = 1024
    xs0, dinv = pl.pallas_call(
        _pre_body,
        grid=(NH // BRP,),
        in_specs=[
            pl.BlockSpec((BRP, D), lambda i: (i, 0)),
            pl.BlockSpec((D, H), lambda i: (0, 0)),
            pl.BlockSpec((NW, BRP), lambda i: (0, i)),
        ],
        out_specs=[
            pl.BlockSpec((BRP, H), lambda i: (i, 0)),
            pl.BlockSpec((BRP, 1), lambda i: (i, 0)),
        ],
        out_shape=[
            jax.ShapeDtypeStruct((N, H), jnp.float32),
            jax.ShapeDtypeStruct((N, 1), jnp.float32),
        ],
    )(x, W0, degp)

    def mid(parts, xs, b, w):
        return pl.pallas_call(
            _mid_body,
            grid=grid,
            in_specs=[
                pl.BlockSpec((NC, BR, H), lambda i: (0, i, 0)),
                pl.BlockSpec((BR, H), lambda i: (i, 0)),
                pl.BlockSpec((BR, 1), lambda i: (i, 0)),
                pl.BlockSpec((1, H), lambda i: (0, 0)),
                pl.BlockSpec((H, H), lambda i: (0, 0)),
            ],
            out_specs=pl.BlockSpec((BR, H), lambda i: (i, 0)),
            out_shape=jax.ShapeDtypeStruct((N, H), jnp.float32),
        )(parts, xs, dinv, b, w)

    p1 = agg_kernel(xs0, src, dst, zerosH).reshape(NC, NA, H)
    xs1 = mid(p1, xs0, b0r, W1)
    p2 = agg_kernel(xs1, src, dst, zerosH).reshape(NC, NA, H)
    xs2 = mid(p2, xs1, b1r, W2)
    p3 = agg_kernel(xs2, src, dst, zerosH).reshape(NC, NA, H)

    out = pl.pallas_call(
        functools.partial(_final_body, N, C),
        grid=grid,
        in_specs=[
            pl.BlockSpec((NC, BR, H), lambda i: (0, i, 0)),
            pl.BlockSpec((BR, H), lambda i: (i, 0)),
            pl.BlockSpec((BR, 1), lambda i: (i, 0)),
            pl.BlockSpec((1, H), lambda i: (0, 0)),
            pl.BlockSpec((H, 128), lambda i: (0, 0)),
            pl.BlockSpec((1, 128), lambda i: (0, 0)),
        ],
        out_specs=pl.BlockSpec((1, 128), lambda i: (0, 0)),
        out_shape=jax.ShapeDtypeStruct((1, 128), jnp.float32),
        scratch_shapes=[pltpu.VMEM((1, 128), jnp.float32)],
    )(p3, xs2, dinv, b2r, wl_pad, bl_pad)

    return out[:, :C]


# speedup vs baseline: 1.5171x; 1.0782x over previous
import functools

import jax
import jax.numpy as jnp
from jax import lax
from jax.experimental import pallas as pl
from jax.experimental.pallas import tpu as pltpu
from jax.experimental.pallas import tpu_sc as plsc

_K = 80


@functools.lru_cache(maxsize=None)
def _sc_kernels(N, E, H):
    info = plsc.get_sparse_core_info()
    NC, NS = info.num_cores, info.num_subcores
    NW = NC * NS

    assert E % _K == 0, "edge count must be padded to a multiple of 128"
    CH = E // _K
    q, r = divmod(CH, NW)
    NPAIR = q // 2

    NA = -(-(N + 1) // (NS * 8)) * NS * 8
    RPT = NA // NS
    NH = -(-(N + 1) // 1024) * 1024

    mesh = plsc.VectorSubcoreMesh(core_axis_name="c", subcore_axis_name="s")

    @functools.partial(
        pl.kernel,
        mesh=mesh,
        out_type=jax.ShapeDtypeStruct((NC * NA, H), jnp.float32),
        scratch_types=[
            pltpu.VMEM((4, _K), jnp.int32),
            pltpu.VMEM((4, _K), jnp.int32),
            pltpu.VMEM((4, _K, H), jnp.float32),
            pltpu.VMEM_SHARED((NA, H), jnp.float32),
        ] + [pltpu.SemaphoreType.DMA] * 16,
    )
    def agg_kernel(xs, srce, dste, zeros, out, srcb, dstb, rowsb, acc,
                   *sems):
        cid = lax.axis_index("c")
        sid = lax.axis_index("s")
        wid = cid * NS + sid
        nchw = q + jnp.where(wid < r, 1, 0)
        c0 = wid * q + jnp.minimum(wid, r)

        gsem = sems[0:4]
        ssem = sems[4:8]
        dsem = sems[8:12]
        csem = sems[12:16]

        def ebase(j):
            return (c0 + j) * _K

        def load_src(j, m):
            pltpu.async_copy(srce.at[pl.ds(ebase(j), _K)], srcb.at[m],
                             ssem[m])

        def load_dst(j, m):
            pltpu.async_copy(dste.at[pl.ds(ebase(j), _K)], dstb.at[m],
                             dsem[m])

        def wait_src(m):
            pltpu.make_async_copy(srce.at[pl.ds(0, _K)], srcb.at[m],
                                  ssem[m]).wait()

        def wait_dst(m):
            pltpu.make_async_copy(dste.at[pl.ds(0, _K)], dstb.at[m],
                                  dsem[m]).wait()

        def gather(m):
            pltpu.async_copy(xs.at[srcb.at[m]], rowsb.at[m], gsem[m])

        def wait_gather(m):
            pltpu.make_async_copy(xs.at[srcb.at[m]], rowsb.at[m],
                                  gsem[m]).wait()

        def scatter(m):
            pltpu.async_copy(rowsb.at[m], acc.at[dstb.at[m]], csem[m],
                             add=True)

        def wait_scatter(m):
            pltpu.make_async_copy(rowsb.at[m], acc.at[dstb.at[m]],
                                  csem[m]).wait()

        for jj in range(4):
            load_src(jj, jj)
            load_dst(jj, jj)
        wait_src(0)
        gather(0)
        wait_src(1)
        gather(1)
        pltpu.sync_copy(zeros.at[pl.ds(sid * RPT, RPT)],
                        acc.at[pl.ds(sid * RPT, RPT)])
        plsc.subcore_barrier()

        def chunk_body(j, m):
            m2 = (m + 2) % 4

            wait_gather(m)
            wait_dst(m)
            scatter(m)

            @pl.when(j + 2 < nchw)
            def _():
                @pl.when(j >= 2)
                def _():
                    wait_scatter(m2)
                    load_dst(j + 2, m2)
                wait_src(m2)
                gather(m2)

            @pl.when(j + 4 < nchw)
            def _():
                load_src(j + 4, m)

        def chunk4(jq, carry):
            j = 4 * jq
            chunk_body(j, 0)
            chunk_body(j + 1, 1)
            chunk_body(j + 2, 2)
            chunk_body(j + 3, 3)
            return carry

        lax.fori_loop(0, q // 4, chunk4, 0)
        for j in range(q - q % 4, q):
            chunk_body(j, j % 4)
        if r:
            @pl.when(wid < r)
            def _():
                chunk_body(q, q % 4)

        for m in range(4):
            wait_scatter(m)
        plsc.subcore_barrier()
        pltpu.sync_copy(acc.at[pl.ds(sid * RPT, RPT)],
                        out.at[pl.ds(cid * NA + sid * RPT, RPT)])

    @functools.partial(
        pl.kernel,
        mesh=mesh,
        out_type=jax.ShapeDtypeStruct((NW * NH,), jnp.float32),
        compiler_params=pltpu.CompilerParams(needs_layout_passes=False),
        scratch_types=[
            pltpu.VMEM((2, _K), jnp.int32),
            pltpu.VMEM((NH,), jnp.float32),
            pltpu.SemaphoreType.DMA,
            pltpu.SemaphoreType.DMA,
        ],
    )
    def hist_kernel(dste, zeros1, out, dstb, hist, sd0, sd1):
        cid = lax.axis_index("c")
        sid = lax.axis_index("s")
        wid = cid * NS + sid
        nchw = q + jnp.where(wid < r, 1, 0)
        c0 = wid * q + jnp.minimum(wid, r)

        pltpu.sync_copy(zeros1, hist)

        dsem = (sd0, sd1)

        def load_dst(j, s):
            pltpu.async_copy(dste.at[pl.ds((c0 + j) * _K, _K)], dstb.at[s],
                             dsem[s])

        def wait_dst(s):
            pltpu.make_async_copy(dste.at[pl.ds(0, _K)], dstb.at[s],
                                  dsem[s]).wait()

        lanes = lax.iota(jnp.int32, 16)
        ones16 = jnp.ones((16,), jnp.float32)

        def count16(d):
            for k in range(16):
                plsc.addupdate_scatter(hist, [d], ones16, mask=lanes == k)

        def chunk_body(j, s):
            wait_dst(s)
            for v in range(_K // 16):
                count16(dstb[s, pl.ds(v * 16, 16)])

            @pl.when(j + 2 < nchw)
            def _():
                load_dst(j + 2, s)

        load_dst(0, 0)
        load_dst(1, 1)

        def pair(jp, carry):
            chunk_body(2 * jp, 0)
            chunk_body(2 * jp + 1, 1)
            return carry

        lax.fori_loop(0, NPAIR, pair, 0)
        if q % 2:
            chunk_body(q - 1, (q - 1) % 2)
        if r:
            @pl.when(wid < r)
            def _():
                chunk_body(q, q % 2)

        pltpu.sync_copy(hist, out.at[pl.ds(wid * NH, NH)])

    return agg_kernel, hist_kernel, NA, NH, NC, NW


def _pre_body(x_ref, w_ref, degp_ref, xs_ref, dinv_ref):
    deg = 1.0 + jnp.sum(degp_ref[...], axis=0)
    dinv = lax.rsqrt(deg).reshape(-1, 1)
    dinv_ref[...] = dinv
    xs_ref[...] = dinv * jnp.dot(x_ref[...], w_ref[...],
                                 preferred_element_type=jnp.float32)


def _mid_body(p_ref, xs_ref, dinv_ref, b_ref, w_ref, o_ref):
    dinv = dinv_ref[...]
    h = jnp.maximum(dinv * (p_ref[0] + p_ref[1] + xs_ref[...]) + b_ref[...],
                    0.0)
    o_ref[...] = dinv * jnp.dot(h, w_ref[...],
                                preferred_element_type=jnp.float32)


def _final_body(n_nodes, ncols, p_ref, xs_ref, dinv_ref, b_ref, wl_ref,
                bl_ref, o_ref, acc_ref):
    i = pl.program_id(0)

    @pl.when(i == 0)
    def _():
        acc_ref[...] = jnp.zeros_like(acc_ref)

    h = dinv_ref[...] * (p_ref[0] + p_ref[1] + xs_ref[...]) + b_ref[...]
    acc_ref[...] += jnp.sum(h, axis=0, keepdims=True)

    @pl.when(i == pl.num_programs(0) - 1)
    def _():
        pooled = acc_ref[...] * (1.0 / n_nodes)
        logits = jnp.dot(pooled, wl_ref[...],
                         preferred_element_type=jnp.float32) + bl_ref[...]
        col = lax.broadcasted_iota(jnp.int32, logits.shape, 1)
        valid = col < ncols
        mx = jnp.max(jnp.where(valid, logits, -jnp.inf), axis=1,
                     keepdims=True)
        ez = jnp.where(valid, jnp.exp(logits - mx), 0.0)
        o_ref[...] = ez / jnp.sum(ez, axis=1, keepdims=True)


def kernel(x, edge_index, W0, b0, W1, b1, W2, b2, Wlin, blin):
    N, D = x.shape
    H = W0.shape[1]
    C = Wlin.shape[1]
    E = edge_index.shape[1]
    src = edge_index[0]
    dst = edge_index[1]
    if E % _K:
        pad = _K - E % _K
        src = jnp.concatenate([src, jnp.zeros((pad,), src.dtype)])
        dst = jnp.concatenate([dst, jnp.full((pad,), N, dst.dtype)])
        E += pad

    agg_kernel, hist_kernel, NA, NH, NC, NW = _sc_kernels(N, E, H)

    zerosH = jnp.zeros((NA, H), jnp.float32)
    zeros1 = jnp.zeros((NH,), jnp.float32)

    degp = hist_kernel(dst, zeros1).reshape(NW, NH)

    BR = 1000 if N % 1000 == 0 else 8
    grid = (N // BR,)
    b0r, b1r, b2r = (b.reshape(1, H) for b in (b0, b1, b2))
    wl_pad = jnp.zeros((H, 128), jnp.float32).at[:, :C].set(Wlin)
    bl_pad = jnp.zeros((1, 128), jnp.float32).at[:, :C].set(blin)

    BRP = 1024
    xs0, dinv = pl.pallas_call(
        _pre_body,
        grid=(NH // BRP,),
        in_specs=[
            pl.BlockSpec((BRP, D), lambda i: (i, 0)),
            pl.BlockSpec((D, H), lambda i: (0, 0)),
            pl.BlockSpec((NW, BRP), lambda i: (0, i)),
        ],
        out_specs=[
            pl.BlockSpec((BRP, H), lambda i: (i, 0)),
            pl.BlockSpec((BRP, 1), lambda i: (i, 0)),
        ],
        out_shape=[
            jax.ShapeDtypeStruct((N, H), jnp.float32),
            jax.ShapeDtypeStruct((N, 1), jnp.float32),
        ],
    )(x, W0, degp)

    def mid(parts, xs, b, w):
        return pl.pallas_call(
            _mid_body,
            grid=grid,
            in_specs=[
                pl.BlockSpec((NC, BR, H), lambda i: (0, i, 0)),
                pl.BlockSpec((BR, H), lambda i: (i, 0)),
                pl.BlockSpec((BR, 1), lambda i: (i, 0)),
                pl.BlockSpec((1, H), lambda i: (0, 0)),
                pl.BlockSpec((H, H), lambda i: (0, 0)),
            ],
            out_specs=pl.BlockSpec((BR, H), lambda i: (i, 0)),
            out_shape=jax.ShapeDtypeStruct((N, H), jnp.float32),
        )(parts, xs, dinv, b, w)

    p1 = agg_kernel(xs0, src, dst, zerosH).reshape(NC, NA, H)
    xs1 = mid(p1, xs0, b0r, W1)
    p2 = agg_kernel(xs1, src, dst, zerosH).reshape(NC, NA, H)
    xs2 = mid(p2, xs1, b1r, W2)
    p3 = agg_kernel(xs2, src, dst, zerosH).reshape(NC, NA, H)

    out = pl.pallas_call(
        functools.partial(_final_body, N, C),
        grid=grid,
        in_specs=[
            pl.BlockSpec((NC, BR, H), lambda i: (0, i, 0)),
            pl.BlockSpec((BR, H), lambda i: (i, 0)),
            pl.BlockSpec((BR, 1), lambda i: (i, 0)),
            pl.BlockSpec((1, H), lambda i: (0, 0)),
            pl.BlockSpec((H, 128), lambda i: (0, 0)),
            pl.BlockSpec((1, 128), lambda i: (0, 0)),
        ],
        out_specs=pl.BlockSpec((1, 128), lambda i: (0, 0)),
        out_shape=jax.ShapeDtypeStruct((1, 128), jnp.float32),
        scratch_shapes=[pltpu.VMEM((1, 128), jnp.float32)],
    )(p3, xs2, dinv, b2r, wl_pad, bl_pad)

    return out[:, :C]
